# baseline (speedup 1.0000x reference)
import jax
import jax.numpy as jnp
from jax.experimental import pallas as pl
from jax.experimental.pallas import tpu as pltpu

CLASS_NUM = 80
CONF_THRE = 0.2
NMS_THRE = 0.45

N = 5000
NPAD = 5120
ROWS = 8
COLS = 640
RB = 8


def _argmax_bcast(v, i):
    def lex(va, ia, vb, ib):
        take = (vb > va) | ((vb == va) & (ib < ia))
        return jnp.where(take, vb, va), jnp.where(take, ib, ia)

    v01, i01 = lex(v[:, 0:128], i[:, 0:128], v[:, 128:256], i[:, 128:256])
    v23, i23 = lex(v[:, 256:384], i[:, 256:384], v[:, 384:512], i[:, 384:512])
    va, ia = lex(v01, i01, v23, i23)
    va, ia = lex(va, ia, v[:, 512:640], i[:, 512:640])
    for sh in (1, 2, 4, 8, 16, 32, 64):
        va, ia = lex(va, ia, pltpu.roll(va, sh, 1), pltpu.roll(ia, sh, 1))
    for sh in (1, 2, 4):
        va, ia = lex(va, ia, pltpu.roll(va, sh, 0), pltpu.roll(ia, sh, 0))
    vb = jnp.concatenate([va] * 5, axis=1)
    ib = jnp.concatenate([ia] * 5, axis=1)
    return vb, ib


def _dm_kernel(pred_ref, pred4_ref, zone_ref,
               y1o, x1o, y2o, x2o, inzko, sco, clso, cyo, cxo, keepo,
               x1r, y1r, x2r, y2r, arear, iotar):
    f32 = jnp.float32
    cx = pred_ref[0]
    cy = pred_ref[1]
    w = pred_ref[2]
    h = pred_ref[3]
    obj = pred_ref[4]
    x1r[...] = cx - w / 2
    y1r[...] = cy - h / 2
    x2r[...] = cx + w / 2
    y2r[...] = cy + h / 2
    arear[...] = (jnp.clip(x2r[...] - x1r[...], 0.0)
                  * jnp.clip(y2r[...] - y1r[...], 0.0))
    iotar[...] = (jax.lax.broadcasted_iota(jnp.int32, (ROWS, COLS), 0) * COLS
                  + jax.lax.broadcasted_iota(jnp.int32, (ROWS, COLS), 1))

    def cbody(k, carry):
        best, bk = carry
        v = pred_ref[5 + k]
        gt = v > best
        return jnp.where(gt, v, best), jnp.where(gt, k, bk)

    best0 = pred_ref[5]
    bk0 = jnp.zeros((ROWS, COLS), jnp.int32)
    class_conf, class_pred = jax.lax.fori_loop(1, CLASS_NUM, cbody, (best0, bk0))

    snms = obj * class_conf
    valid = snms >= CONF_THRE
    s0 = jnp.where(valid, snms, f32(-1.0))
    m0 = jnp.max(s0)

    def nms_cond(carry):
        _, m = carry
        return m >= CONF_THRE

    def nms_body(carry):
        s, _ = carry
        iota = iotar[...]
        x1 = x1r[...]
        y1 = y1r[...]
        x2 = x2r[...]
        y2 = y2r[...]
        area = arear[...]
        ninf = f32(-jnp.inf)

        sels = []
        valids = []
        coords = []
        s_cur = s
        for _k in range(RB):
            mkv, ikv = _argmax_bcast(s_cur, iota)
            sel_k = iota == ikv
            s_cur = jnp.where(sel_k, f32(-1.0), s_cur)
            i_k = ikv[0, 0]
            cxs = pred4_ref[0, i_k]
            cys = pred4_ref[1, i_k]
            ws = pred4_ref[2, i_k]
            hs = pred4_ref[3, i_k]
            x1s = cxs - ws / 2
            y1s = cys - hs / 2
            x2s = cxs + ws / 2
            y2s = cys + hs / 2
            areas = jnp.clip(x2s - x1s, 0.0) * jnp.clip(y2s - y1s, 0.0)
            sels.append(sel_k)
            valids.append(mkv[0, 0] >= CONF_THRE)
            coords.append((x1s, y1s, x2s, y2s, areas))

        accs = []
        for _k in range(RB):
            x1k, y1k, x2k, y2k, ak = coords[_k]
            ok = valids[_k]
            for _j in range(_k):
                x1j, y1j, x2j, y2j, aj = coords[_j]
                ltx = jnp.maximum(x1j, x1k)
                lty = jnp.maximum(y1j, y1k)
                rbx = jnp.minimum(x2j, x2k)
                rby = jnp.minimum(y2j, y2k)
                inter = jnp.clip(rbx - ltx, 0.0) * jnp.clip(rby - lty, 0.0)
                union = aj + ak - inter
                iou = inter / jnp.maximum(union, f32(1e-9))
                ok = ok & jnp.logical_not(accs[_j] & (iou > NMS_THRE))
            accs.append(ok)

        supp = jnp.zeros((ROWS, COLS), jnp.bool_)
        selacc = jnp.zeros((ROWS, COLS), jnp.bool_)
        for _k in range(RB):
            x1s, y1s, x2s, y2s, areas = coords[_k]
            ltx = jnp.maximum(x1s, x1)
            lty = jnp.maximum(y1s, y1)
            rbx = jnp.minimum(x2s, x2)
            rby = jnp.minimum(y2s, y2)
            iw = jnp.clip(rbx - ltx, 0.0)
            ih = jnp.clip(rby - lty, 0.0)
            inter = iw * ih
            union = areas + area - inter
            iou = inter / jnp.maximum(union, f32(1e-9))
            supp = supp | ((iou > NMS_THRE) & accs[_k])
            selacc = selacc | (sels[_k] & accs[_k])

        s2 = jnp.where(selacc, f32(-2.0), jnp.where(supp, f32(-1.0), s))
        return s2, jnp.max(s2)

    sf, _ = jax.lax.while_loop(nms_cond, nms_body, (s0, m0))
    keepb = sf == f32(-2.0)
    mk = jnp.where(keepb, f32(1.0), f32(0.0))

    x1 = x1r[...]
    y1 = y1r[...]
    x2 = x2r[...]
    y2 = y2r[...]
    px = (x1 + x2) / 2
    py = (y1 + y2) / 2

    parity = jnp.zeros((ROWS, COLS), jnp.bool_)
    for k in range(8):
        xi = zone_ref[k, 0]
        yi = zone_ref[k, 1]
        xj = zone_ref[(k - 1) % 8, 0]
        yj = zone_ref[(k - 1) % 8, 1]
        gyi = yi > py
        gyj = yj > py
        gx = (xj - xi) * (py - yi) / (yj - yi) + xi
        parity = parity ^ ((gyi != gyj) & (gx > px))

    y1o[...] = y1 * mk
    x1o[...] = x1 * mk
    y2o[...] = y2 * mk
    x2o[...] = x2 * mk
    inzko[...] = (parity & keepb).astype(jnp.int32)
    sco[...] = jnp.maximum(obj, class_conf) * mk
    clso[...] = jnp.where(keepb, class_pred, -1)
    cyo[...] = py * mk
    cxo[...] = px * mk
    keepo[...] = keepb.astype(jnp.int32)


def kernel(prediction, zone):
    p = prediction[0]
    pT = jnp.pad(jnp.transpose(p), ((0, 0), (0, NPAD - N)))
    pp = pT.reshape(85, ROWS, COLS)
    pred4 = pT[:4]

    f32 = jnp.float32
    outs = pl.pallas_call(
        _dm_kernel,
        in_specs=[
            pl.BlockSpec(memory_space=pltpu.VMEM),
            pl.BlockSpec(memory_space=pltpu.SMEM),
            pl.BlockSpec(memory_space=pltpu.SMEM),
        ],
        out_shape=[
            jax.ShapeDtypeStruct((ROWS, COLS), f32),
            jax.ShapeDtypeStruct((ROWS, COLS), f32),
            jax.ShapeDtypeStruct((ROWS, COLS), f32),
            jax.ShapeDtypeStruct((ROWS, COLS), f32),
            jax.ShapeDtypeStruct((ROWS, COLS), jnp.int32),
            jax.ShapeDtypeStruct((ROWS, COLS), f32),
            jax.ShapeDtypeStruct((ROWS, COLS), jnp.int32),
            jax.ShapeDtypeStruct((ROWS, COLS), f32),
            jax.ShapeDtypeStruct((ROWS, COLS), f32),
            jax.ShapeDtypeStruct((ROWS, COLS), jnp.int32),
        ],
        scratch_shapes=[
            pltpu.VMEM((ROWS, COLS), f32),
            pltpu.VMEM((ROWS, COLS), f32),
            pltpu.VMEM((ROWS, COLS), f32),
            pltpu.VMEM((ROWS, COLS), f32),
            pltpu.VMEM((ROWS, COLS), f32),
            pltpu.VMEM((ROWS, COLS), jnp.int32),
        ],
    )(pp, pred4, zone)

    y1m, x1m, y2m, x2m, inzk, sc, cls_o, cym, cxm, keep = [
        o.reshape(NPAD)[:N] for o in outs
    ]
    boxes_yxyx = jnp.stack([y1m, x1m, y2m, x2m], axis=1)
    centers_yx = jnp.stack([cym, cxm], axis=1)
    return (boxes_yxyx,
            inzk.astype(jnp.bool_),
            sc,
            cls_o,
            centers_yx,
            keep.astype(jnp.bool_))


# speedup vs baseline: 1.4810x; 1.4810x over previous
import functools

import jax
import jax.numpy as jnp
from jax import lax
from jax.experimental import pallas as pl
from jax.experimental.pallas import tpu as pltpu
from jax.experimental.pallas import tpu_sc as plsc

CLASS_NUM = 80
CONF_THRE = 0.2
NMS_THRE = 0.45

N = 5000
NPAD = 5120
ROWS = 8
COLS = 640

SC_NW = 32
SC_PER_W = NPAD // SC_NW
SC_LANES = 16


def _zone_sc_kernel(cx_hbm, cy_hbm, w_hbm, h_hbm, zone_hbm, out_hbm,
                    cxv, cyv, wv, hv, zonev, outv):
    wid = lax.axis_index("s") * 2 + lax.axis_index("c")
    base = wid * SC_PER_W
    pltpu.sync_copy(cx_hbm.at[pl.ds(base, SC_PER_W)], cxv)
    pltpu.sync_copy(cy_hbm.at[pl.ds(base, SC_PER_W)], cyv)
    pltpu.sync_copy(w_hbm.at[pl.ds(base, SC_PER_W)], wv)
    pltpu.sync_copy(h_hbm.at[pl.ds(base, SC_PER_W)], hv)
    pltpu.sync_copy(zone_hbm, zonev)

    edges = []
    for k in range(8):
        xi = zonev[pl.ds((k * 4 + 0) * SC_LANES, SC_LANES)]
        yi = zonev[pl.ds((k * 4 + 1) * SC_LANES, SC_LANES)]
        xj = zonev[pl.ds((k * 4 + 2) * SC_LANES, SC_LANES)]
        yj = zonev[pl.ds((k * 4 + 3) * SC_LANES, SC_LANES)]
        edges.append((xi, yi, xj, yj))

    for j in range(SC_PER_W // SC_LANES):
        sl = pl.ds(j * SC_LANES, SC_LANES)
        cx = cxv[sl]
        cy = cyv[sl]
        w = wv[sl]
        h = hv[sl]
        x1 = cx - w / 2
        x2 = cx + w / 2
        y1 = cy - h / 2
        y2 = cy + h / 2
        px = (x1 + x2) / 2
        py = (y1 + y2) / 2
        one = jnp.ones((SC_LANES,), jnp.int32)
        zero = jnp.zeros((SC_LANES,), jnp.int32)
        parity = zero
        for (xi, yi, xj, yj) in edges:
            gyi = jnp.where(yi > py, one, zero)
            gyj = jnp.where(yj > py, one, zero)
            gx = (xj - xi) * (py - yi) / (yj - yi) + xi
            gpx = jnp.where(gx > px, one, zero)
            parity = parity ^ ((gyi ^ gyj) & gpx)
        outv[sl] = parity

    pltpu.sync_copy(outv, out_hbm.at[pl.ds(base, SC_PER_W)])


def _zone_sc(pT, zone):
    krn = functools.partial(
        pl.kernel,
        mesh=plsc.VectorSubcoreMesh(core_axis_name="c", subcore_axis_name="s"),
        out_type=jax.ShapeDtypeStruct((NPAD,), jnp.int32),
        scratch_types=[
            pltpu.VMEM((SC_PER_W,), jnp.float32),
            pltpu.VMEM((SC_PER_W,), jnp.float32),
            pltpu.VMEM((SC_PER_W,), jnp.float32),
            pltpu.VMEM((SC_PER_W,), jnp.float32),
            pltpu.VMEM((8 * 4 * SC_LANES,), jnp.float32),
            pltpu.VMEM((SC_PER_W,), jnp.int32),
        ],
    )(_zone_sc_kernel)
    zs = jnp.roll(zone, shift=1, axis=0)
    zedges = jnp.stack([zone[:, 0], zone[:, 1], zs[:, 0], zs[:, 1]], axis=1)
    zbcast = jnp.broadcast_to(zedges[:, :, None], (8, 4, SC_LANES)).reshape(-1)
    return krn(pT[0], pT[1], pT[2], pT[3], zbcast)


def _dm_kernel(pred_ref, pred4_ref,
               y1o, x1o, y2o, x2o, sco, clso, cyo, cxo, keepo,
               x1r, y1r, x2r, y2r, arear, iotar):
    f32 = jnp.float32
    cx = pred_ref[0]
    cy = pred_ref[1]
    w = pred_ref[2]
    h = pred_ref[3]
    obj = pred_ref[4]
    x1r[...] = cx - w / 2
    y1r[...] = cy - h / 2
    x2r[...] = cx + w / 2
    y2r[...] = cy + h / 2
    arear[...] = (jnp.clip(x2r[...] - x1r[...], 0.0)
                  * jnp.clip(y2r[...] - y1r[...], 0.0))
    iotar[...] = (jax.lax.broadcasted_iota(jnp.int32, (ROWS, COLS), 0) * COLS
                  + jax.lax.broadcasted_iota(jnp.int32, (ROWS, COLS), 1))

    def cbody(k, carry):
        best, bk = carry
        v = pred_ref[5 + k]
        gt = v > best
        return jnp.where(gt, v, best), jnp.where(gt, k, bk)

    best0 = pred_ref[5]
    bk0 = jnp.zeros((ROWS, COLS), jnp.int32)
    class_conf, class_pred = jax.lax.fori_loop(1, CLASS_NUM, cbody, (best0, bk0))

    snms = obj * class_conf
    valid = snms >= CONF_THRE
    s0 = jnp.where(valid, snms, f32(-1.0))
    m0 = jnp.max(s0)

    def nms_cond(carry):
        _, m = carry
        return m >= CONF_THRE

    def nms_body(carry):
        s, m = carry
        iota = iotar[...]
        i = jnp.min(jnp.where(s == m, iota, jnp.int32(NPAD)))
        cxs = pred4_ref[0, i]
        cys = pred4_ref[1, i]
        ws = pred4_ref[2, i]
        hs = pred4_ref[3, i]
        x1s = cxs - ws / 2
        y1s = cys - hs / 2
        x2s = cxs + ws / 2
        y2s = cys + hs / 2
        areas = jnp.clip(x2s - x1s, 0.0) * jnp.clip(y2s - y1s, 0.0)
        ltx = jnp.maximum(x1s, x1r[...])
        lty = jnp.maximum(y1s, y1r[...])
        rbx = jnp.minimum(x2s, x2r[...])
        rby = jnp.minimum(y2s, y2r[...])
        iw = jnp.clip(rbx - ltx, 0.0)
        ih = jnp.clip(rby - lty, 0.0)
        inter = iw * ih
        union = areas + arear[...] - inter
        iou = inter / jnp.maximum(union, f32(1e-9))
        sel = iota == i
        s2 = jnp.where(sel, f32(-2.0), jnp.where(iou > NMS_THRE, f32(-1.0), s))
        return s2, jnp.max(s2)

    sf, _ = jax.lax.while_loop(nms_cond, nms_body, (s0, m0))
    keepb = sf == f32(-2.0)
    mk = jnp.where(keepb, f32(1.0), f32(0.0))

    x1 = x1r[...]
    y1 = y1r[...]
    x2 = x2r[...]
    y2 = y2r[...]
    px = (x1 + x2) / 2
    py = (y1 + y2) / 2

    y1o[...] = y1 * mk
    x1o[...] = x1 * mk
    y2o[...] = y2 * mk
    x2o[...] = x2 * mk
    sco[...] = jnp.maximum(obj, class_conf) * mk
    clso[...] = jnp.where(keepb, class_pred, -1)
    cyo[...] = py * mk
    cxo[...] = px * mk
    keepo[...] = keepb.astype(jnp.int32)


def kernel(prediction, zone):
    p = prediction[0]
    pT = jnp.pad(jnp.transpose(p), ((0, 0), (0, NPAD - N)))
    pp = pT.reshape(85, ROWS, COLS)
    pred4 = pT[:4]

    f32 = jnp.float32
    outs = pl.pallas_call(
        _dm_kernel,
        in_specs=[
            pl.BlockSpec(memory_space=pltpu.VMEM),
            pl.BlockSpec(memory_space=pltpu.SMEM),
        ],
        out_shape=[
            jax.ShapeDtypeStruct((ROWS, COLS), f32),
            jax.ShapeDtypeStruct((ROWS, COLS), f32),
            jax.ShapeDtypeStruct((ROWS, COLS), f32),
            jax.ShapeDtypeStruct((ROWS, COLS), f32),
            jax.ShapeDtypeStruct((ROWS, COLS), f32),
            jax.ShapeDtypeStruct((ROWS, COLS), jnp.int32),
            jax.ShapeDtypeStruct((ROWS, COLS), f32),
            jax.ShapeDtypeStruct((ROWS, COLS), f32),
            jax.ShapeDtypeStruct((ROWS, COLS), jnp.int32),
        ],
        scratch_shapes=[
            pltpu.VMEM((ROWS, COLS), f32),
            pltpu.VMEM((ROWS, COLS), f32),
            pltpu.VMEM((ROWS, COLS), f32),
            pltpu.VMEM((ROWS, COLS), f32),
            pltpu.VMEM((ROWS, COLS), f32),
            pltpu.VMEM((ROWS, COLS), jnp.int32),
        ],
    )(pp, pred4)

    in_zone = _zone_sc(pT, zone)

    y1m, x1m, y2m, x2m, sc, cls_o, cym, cxm, keep = [
        o.reshape(NPAD)[:N] for o in outs
    ]
    keep = keep.astype(jnp.bool_)
    boxes_yxyx = jnp.stack([y1m, x1m, y2m, x2m], axis=1)
    centers_yx = jnp.stack([cym, cxm], axis=1)
    return (boxes_yxyx,
            in_zone[:N].astype(jnp.bool_) & keep,
            sc,
            cls_o,
            centers_yx,
            keep)
